# Initial kernel scaffold; baseline (speedup 1.0000x reference)
import jax
import jax.numpy as jnp
from jax.experimental import pallas as pl


def kernel(x, W, b):
    raise NotImplementedError("write your pallas kernel here")


# speedup vs baseline: 4.4499x; 4.4499x over previous
import functools

import jax
import jax.numpy as jnp
from jax.experimental import pallas as pl
from jax.experimental.pallas import tpu as pltpu

DIM = 4096
E = 64
TOPK = 8
N = 8192
CAPACITY = int(1.0 * N)
EPS = 1e-06

BLK = 512


def _gate_pass1(x_ref, wt_ref, b_ref, masked_ref, acc_ref):
    i = pl.program_id(0)

    logits = jnp.dot(x_ref[...], wt_ref[...],
                     preferred_element_type=jnp.float32) + b_ref[...]
    m = jnp.max(logits, axis=-1, keepdims=True)
    ex = jnp.exp(logits - m)
    gate = ex / jnp.sum(ex, axis=-1, keepdims=True)

    iota = jax.lax.broadcasted_iota(jnp.int32, gate.shape, 1)
    scores = gate
    mask = jnp.zeros_like(gate)
    for _ in range(TOPK):
        mx = jnp.max(scores, axis=-1, keepdims=True)
        idx = jnp.min(jnp.where(scores == mx, iota, E), axis=-1,
                      keepdims=True)
        sel = (iota == idx).astype(jnp.float32)
        mask = mask + sel
        scores = jnp.where(sel > 0, -jnp.inf, scores)

    masked = gate * mask
    masked_ref[...] = masked

    part = jnp.concatenate(
        [jnp.sum(masked, axis=0, keepdims=True),
         jnp.sum(mask, axis=0, keepdims=True),
         jnp.sum(gate, axis=0, keepdims=True),
         jnp.zeros((5, E), jnp.float32)], axis=0)

    @pl.when(i == 0)
    def _init():
        acc_ref[...] = part

    @pl.when(i > 0)
    def _accum():
        acc_ref[...] += part


def _gate_pass2(masked_ref, acc_ref, out_ref, loss_ref):
    denom = acc_ref[0:1, :] + EPS
    out_ref[...] = masked_ref[...] / denom * float(CAPACITY)
    density = acc_ref[1:2, :] * (1.0 / N)
    proxy = acc_ref[2:3, :] * (1.0 / N)
    loss_ref[0, 0] = jnp.sum(density * proxy) * (float(E) ** 2 / E)


@jax.jit
def kernel(x, W, b):
    wt = W.T
    b2 = b.reshape(1, E)
    nblk = N // BLK

    masked, acc = pl.pallas_call(
        _gate_pass1,
        grid=(nblk,),
        in_specs=[
            pl.BlockSpec((BLK, DIM), lambda i: (i, 0)),
            pl.BlockSpec((DIM, E), lambda i: (0, 0)),
            pl.BlockSpec((1, E), lambda i: (0, 0)),
        ],
        out_specs=[
            pl.BlockSpec((BLK, E), lambda i: (i, 0)),
            pl.BlockSpec((8, E), lambda i: (0, 0)),
        ],
        out_shape=[
            jax.ShapeDtypeStruct((N, E), jnp.float32),
            jax.ShapeDtypeStruct((8, E), jnp.float32),
        ],
    )(x, wt, b2)

    out, loss = pl.pallas_call(
        _gate_pass2,
        in_specs=[
            pl.BlockSpec((N, E), lambda: (0, 0)),
            pl.BlockSpec((8, E), lambda: (0, 0)),
        ],
        out_specs=[
            pl.BlockSpec((N, E), lambda: (0, 0)),
            pl.BlockSpec((1, 1), lambda: (0, 0), memory_space=pltpu.SMEM),
        ],
        out_shape=[
            jax.ShapeDtypeStruct((N, E), jnp.float32),
            jax.ShapeDtypeStruct((1, 1), jnp.float32),
        ],
    )(masked, acc)

    return out, loss[0, 0]
